# baseline (speedup 1.0000x reference)
import functools

import jax
import jax.numpy as jnp
from jax import lax
from jax.experimental import pallas as pl
from jax.experimental.pallas import tpu as pltpu
from jax.experimental.pallas import tpu_sc as plsc


@functools.lru_cache(maxsize=None)
def _make_sc_gather(B, S, SP, V, D, NBUF):
    info = plsc.get_sparse_core_info()
    NC, NS = info.num_cores, info.num_subcores
    NW = NC * NS
    assert B % NW == 0 and SP % 8 == 0
    b_per_w = B // NW
    assert b_per_w % NBUF == 0 and b_per_w >= NBUF >= 2
    mesh = plsc.VectorSubcoreMesh(core_axis_name="c", subcore_axis_name="s")

    @functools.partial(
        pl.kernel,
        mesh=mesh,
        compiler_params=pltpu.CompilerParams(use_tc_tiling_on_sc=False),
        out_type=jax.ShapeDtypeStruct((B, S, D), jnp.float32),
        scratch_types=(
            [pltpu.VMEM((b_per_w * SP,), jnp.int32)]
            + [pltpu.VMEM((SP, D), jnp.float32) for _ in range(NBUF)]
            + [pltpu.SemaphoreType.DMA for _ in range(2 * NBUF)]
        ),
    )
    def gather_kernel(table_hbm, idx_hbm, out_hbm, idx_v, *rest):
        bufs = rest[:NBUF]
        gsems = rest[NBUF:2 * NBUF]
        ssems = rest[2 * NBUF:3 * NBUF]
        wid = lax.axis_index("s") * NC + lax.axis_index("c")
        base = wid * b_per_w
        pltpu.sync_copy(idx_hbm.at[pl.ds(base * SP, b_per_w * SP)], idx_v)

        def start_gather(k, s):
            pltpu.async_copy(
                table_hbm.at[idx_v.at[pl.ds(k * SP, SP)]], bufs[s], gsems[s])

        def wait_gather(s):
            pltpu.make_async_copy(
                table_hbm.at[idx_v.at[pl.ds(0, SP)]], bufs[s], gsems[s]).wait()

        def start_scatter(k, s):
            pltpu.async_copy(
                bufs[s].at[pl.ds(0, S)], out_hbm.at[base + k], ssems[s])

        def wait_scatter(s):
            pltpu.make_async_copy(
                bufs[s].at[pl.ds(0, S)], out_hbm.at[base], ssems[s]).wait()

        for j in range(NBUF - 1):
            start_gather(j, j)

        def group_body(g, carry):
            for b in range(NBUF):
                k = g * NBUF + b
                pb = (b - 1) % NBUF

                @pl.when(k + NBUF - 1 < b_per_w)
                def _():
                    @pl.when(k >= 1)
                    def _():
                        wait_scatter(pb)

                    start_gather(k + NBUF - 1, pb)

                wait_gather(b)
                start_scatter(k, b)
            return carry

        lax.fori_loop(0, b_per_w // NBUF, group_body, 0)
        for s in range(NBUF):
            wait_scatter(s)

    return gather_kernel


def kernel(contexts, table):
    B, S = contexts.shape
    V, D = table.shape
    SP = (S + 7) // 8 * 8
    idx = jnp.pad(contexts.astype(jnp.int32), ((0, 0), (0, SP - S)))
    return _make_sc_gather(B, S, SP, V, D, 2)(table, idx.reshape(B * SP))
